# Initial kernel scaffold; baseline (speedup 1.0000x reference)
import jax
import jax.numpy as jnp
from jax.experimental import pallas as pl


def kernel(stimulus_set, table):
    raise NotImplementedError("write your pallas kernel here")


# speedup vs baseline: 7.1724x; 7.1724x over previous
import functools

import jax
import jax.numpy as jnp
from jax import lax
from jax.experimental import pallas as pl
from jax.experimental.pallas import tpu as pltpu
from jax.experimental.pallas import tpu_sc as plsc

B = 16384
NCOL = 5
NREF = 4
NSTIM = 21
NDIM = 3
NC, NS, L = 2, 16, 16
NW = NC * NS
ROWS_W = B // NW
STEPS = ROWS_W // L


def _sqrt16(x):
    i = plsc.bitcast(x, jnp.int32)
    y = plsc.bitcast(jnp.int32(0x5F3759DF) - (i >> 1), jnp.float32)
    for _ in range(3):
        y = y * (1.5 - 0.5 * x * y * y)
    return x * y


_MESH = plsc.VectorSubcoreMesh(core_axis_name="c", subcore_axis_name="s")


@functools.partial(
    pl.kernel,
    mesh=_MESH,
    compiler_params=pltpu.CompilerParams(
        needs_layout_passes=False, use_tc_tiling_on_sc=False),
    out_type=jax.ShapeDtypeStruct((B, NREF), jnp.float32),
    scratch_types=[
        pltpu.VMEM((ROWS_W, NCOL), jnp.int32),
        pltpu.VMEM((NSTIM, NDIM), jnp.float32),
        pltpu.VMEM((ROWS_W, NREF), jnp.float32),
    ],
)
def _rank_sc(ss_hbm, tab_hbm, out_hbm, ss_v, tab_v, out_v):
    wid = lax.axis_index("s") * NC + lax.axis_index("c")
    base = wid * ROWS_W
    pltpu.sync_copy(tab_hbm, tab_v)
    pltpu.sync_copy(ss_hbm.at[pl.ds(base, ROWS_W)], ss_v)
    lanes = lax.iota(jnp.int32, L)
    dcols = [jnp.full((L,), d, jnp.int32) for d in range(NDIM)]

    for v in range(STEPS):
        rb = lanes + v * L
        q = plsc.load_gather(ss_v, [rb, jnp.zeros((L,), jnp.int32)])
        tq = [plsc.load_gather(tab_v, [q, dcols[d]]) for d in range(NDIM)]
        sks = []
        tot = jnp.zeros((L,), jnp.float32)
        for k in range(NREF):
            ck = jnp.full((L,), k + 1, jnp.int32)
            rk = plsc.load_gather(ss_v, [rb, ck])
            acc = jnp.full((L,), 1e-12, jnp.float32)
            for d in range(NDIM):
                df = tq[d] - plsc.load_gather(tab_v, [rk, dcols[d]])
                acc = acc + df * df
            sim = jnp.exp(-10.0 * _sqrt16(acc)) + 0.001
            sk = jnp.where(rk != 0, sim, 0.0)
            sks.append(sk)
            tot = tot + sk
        inv = 1.0 / jnp.maximum(tot, 1e-16)
        for k in range(NREF):
            ck = jnp.full((L,), k, jnp.int32)
            plsc.store_scatter(out_v, [rb, ck], sks[k] * inv)

    pltpu.sync_copy(out_v, out_hbm.at[pl.ds(base, ROWS_W)])


def kernel(stimulus_set, table):
    return _rank_sc(stimulus_set, table)
